# baseline (speedup 1.0000x reference)
import functools

import jax
import jax.numpy as jnp
from jax import lax
from jax.experimental import pallas as pl
from jax.experimental.pallas import tpu as pltpu
from jax.experimental.pallas import tpu_sc as plsc

EMBED = 128
NUM_CTX = 5
LANES = 16
NSLICE = EMBED // LANES
CHUNK = 64


@functools.lru_cache(maxsize=None)
def _make_kernel(B):
    info = plsc.get_sparse_core_info()
    NC, NS = info.num_cores, info.num_subcores
    NW = NC * NS
    items = B // NW
    nchunk = items // CHUNK

    mesh = plsc.VectorSubcoreMesh(core_axis_name="c", subcore_axis_name="s")

    NIDX = 3

    scratch = []
    scratch += [pltpu.VMEM((CHUNK,), jnp.int32) for _ in range(NIDX)]
    scratch += [pltpu.VMEM((CHUNK, NUM_CTX), jnp.int32)
                for _ in range(NIDX)]
    scratch += [pltpu.VMEM((CHUNK,), jnp.int32)
                for _ in range(NUM_CTX * 2)]
    scratch += [pltpu.VMEM((CHUNK, EMBED), jnp.float32)
                for _ in range(2)]
    scratch += [pltpu.VMEM((CHUNK, EMBED), jnp.float32)
                for _ in range(NUM_CTX * 2)]
    scratch.append(pltpu.VMEM((NUM_CTX, items), jnp.float32))
    scratch.append(pltpu.SemaphoreType.DMA)
    scratch.append(pltpu.SemaphoreType.DMA)
    scratch.append(pltpu.SemaphoreType.DMA)

    @functools.partial(
        pl.kernel,
        mesh=mesh,
        out_type=jax.ShapeDtypeStruct((NUM_CTX, B), jnp.float32),
        scratch_types=scratch,
        compiler_params=pltpu.CompilerParams(needs_layout_passes=False),
    )
    def k(tgt_hbm, ctx_hbm, wt_hbm, wc_hbm, out_hbm, *sc):
        it = iter(sc)
        t_gidx = [next(it) for _ in range(NIDX)]
        craw = [next(it) for _ in range(NIDX)]
        c_gidx = [[next(it) for _ in range(2)] for _ in range(NUM_CTX)]
        t_rows = [next(it) for _ in range(2)]
        c_rows = [[next(it) for _ in range(2)] for _ in range(NUM_CTX)]
        out_v = next(it)
        sems = [next(it), next(it)]
        isem = next(it)

        wid = lax.axis_index("s") * NC + lax.axis_index("c")
        wbase = wid * items

        def fetch_idx(g):
            q = g % NIDX
            base = wbase + g * CHUNK
            return [
                pltpu.async_copy(tgt_hbm.at[pl.ds(base, CHUNK)],
                                 t_gidx[q], isem),
                pltpu.async_copy(ctx_hbm.at[pl.ds(base, CHUNK), :],
                                 craw[q], isem),
            ]

        def issue(g, p):
            q = g % NIDX
            for c in range(NUM_CTX):
                ccol = jnp.full((LANES,), c, jnp.int32)
                for b in range(CHUNK // LANES):
                    rowi = b * LANES + lanes
                    c_gidx[c][p][pl.ds(b * LANES, LANES)] = plsc.load_gather(
                        craw[q], [rowi, ccol])
            handles = [pltpu.async_copy(
                wt_hbm.at[t_gidx[q]], t_rows[p], sems[p])]
            for c in range(NUM_CTX):
                handles.append(pltpu.async_copy(
                    wc_hbm.at[c_gidx[c][p]], c_rows[c][p], sems[p]))
            return handles

        lanes = lax.iota(jnp.int32, LANES)

        out_mask = lanes < NUM_CTX

        def compute(g, p):
            @plsc.parallel_loop(0, CHUNK, unroll=2)
            def body(i):
                tr = t_rows[p]
                t = [tr[i, pl.ds(s * LANES, LANES)] for s in range(NSLICE)]
                res = jnp.zeros((LANES,), jnp.float32)
                for c in range(NUM_CTX):
                    cr = c_rows[c][p]
                    acc = t[0] * cr[i, pl.ds(0, LANES)]
                    for s in range(1, NSLICE):
                        acc = acc + t[s] * cr[i, pl.ds(s * LANES, LANES)]
                    d = jnp.sum(acc)
                    res = jnp.where(lanes == c, jnp.full((LANES,), d), res)
                ocol = jnp.full((LANES,), g * CHUNK + i, jnp.int32)
                plsc.store_scatter(out_v, [lanes, ocol], res, mask=out_mask)

        ih = fetch_idx(0)
        for h in ih:
            h.wait()
        handles = issue(0, 0)
        ih_next = fetch_idx(1) if nchunk > 1 else None
        for g in range(nchunk):
            p = g % 2
            nxt = None
            if g + 1 < nchunk:
                for h in ih_next:
                    h.wait()
                nxt = issue(g + 1, 1 - p)
                ih_next = fetch_idx(g + 2) if g + 2 < nchunk else None
            for h in handles:
                h.wait()
            compute(g, p)
            handles = nxt

        for c in range(NUM_CTX):
            pltpu.sync_copy(out_v.at[pl.ds(c, 1), pl.ds(0, items)],
                            out_hbm.at[pl.ds(c, 1), pl.ds(wbase, items)])

    return k


@jax.jit
def kernel(target, context, W_target, W_context):
    B = target.shape[0]
    tgt = target.astype(jnp.int32)
    ctx = context.astype(jnp.int32)
    dots_t = _make_kernel(B)(tgt, ctx, W_target, W_context)
    return jnp.transpose(dots_t)


# speedup vs baseline: 1.0068x; 1.0068x over previous
import functools

import jax
import jax.numpy as jnp
from jax import lax
from jax.experimental import pallas as pl
from jax.experimental.pallas import tpu as pltpu
from jax.experimental.pallas import tpu_sc as plsc

EMBED = 128
NUM_CTX = 5
LANES = 16
NSLICE = EMBED // LANES
CHUNK = 32
STAGES = 3
NIDX = STAGES + 1


@functools.lru_cache(maxsize=None)
def _make_kernel(B):
    info = plsc.get_sparse_core_info()
    NC, NS = info.num_cores, info.num_subcores
    NW = NC * NS
    items = B // NW
    nchunk = items // CHUNK

    mesh = plsc.VectorSubcoreMesh(core_axis_name="c", subcore_axis_name="s")

    scratch = []
    scratch += [pltpu.VMEM((CHUNK,), jnp.int32) for _ in range(NIDX)]
    scratch += [pltpu.VMEM((CHUNK,), jnp.int32)
                for _ in range(NUM_CTX * NIDX)]
    scratch += [pltpu.VMEM((CHUNK, EMBED), jnp.float32)
                for _ in range(STAGES)]
    scratch += [pltpu.VMEM((CHUNK, EMBED), jnp.float32)
                for _ in range(NUM_CTX * STAGES)]
    scratch.append(pltpu.VMEM((NUM_CTX, items), jnp.float32))
    scratch += [pltpu.SemaphoreType.DMA for _ in range(STAGES)]
    scratch.append(pltpu.SemaphoreType.DMA)

    @functools.partial(
        pl.kernel,
        mesh=mesh,
        out_type=jax.ShapeDtypeStruct((NUM_CTX, B), jnp.float32),
        scratch_types=scratch,
        compiler_params=pltpu.CompilerParams(needs_layout_passes=False),
    )
    def k(tgt_hbm, ctx_hbm, wt_hbm, wc_hbm, out_hbm, *sc):
        it = iter(sc)
        t_gidx = [next(it) for _ in range(NIDX)]
        c_gidx = [[next(it) for _ in range(NIDX)] for _ in range(NUM_CTX)]
        t_rows = [next(it) for _ in range(STAGES)]
        c_rows = [[next(it) for _ in range(STAGES)] for _ in range(NUM_CTX)]
        out_v = next(it)
        sems = [next(it) for _ in range(STAGES)]
        isem = next(it)

        wid = lax.axis_index("s") * NC + lax.axis_index("c")
        wbase = wid * items

        def fetch_idx(g):
            q = g % NIDX
            base = wbase + g * CHUNK
            handles = [pltpu.async_copy(
                tgt_hbm.at[pl.ds(base, CHUNK)], t_gidx[q], isem)]
            for c in range(NUM_CTX):
                handles.append(pltpu.async_copy(
                    ctx_hbm.at[c, pl.ds(base, CHUNK)], c_gidx[c][q], isem))
            return handles

        def issue(g):
            q = g % NIDX
            p = g % STAGES
            handles = [pltpu.async_copy(
                wt_hbm.at[t_gidx[q]], t_rows[p], sems[p])]
            for c in range(NUM_CTX):
                handles.append(pltpu.async_copy(
                    wc_hbm.at[c_gidx[c][q]], c_rows[c][p], sems[p]))
            return handles

        lanes = lax.iota(jnp.int32, LANES)
        out_mask = lanes < NUM_CTX

        def compute(g):
            p = g % STAGES

            @plsc.parallel_loop(0, CHUNK, unroll=2)
            def body(i):
                tr = t_rows[p]
                t = [tr[i, pl.ds(s * LANES, LANES)] for s in range(NSLICE)]
                res = jnp.zeros((LANES,), jnp.float32)
                for c in range(NUM_CTX):
                    cr = c_rows[c][p]
                    acc = t[0] * cr[i, pl.ds(0, LANES)]
                    for s in range(1, NSLICE):
                        acc = acc + t[s] * cr[i, pl.ds(s * LANES, LANES)]
                    d = jnp.sum(acc)
                    res = jnp.where(lanes == c, jnp.full((LANES,), d), res)
                ocol = jnp.full((LANES,), g * CHUNK + i, jnp.int32)
                plsc.store_scatter(out_v, [lanes, ocol], res, mask=out_mask)

        AHEAD = STAGES - 1
        ih = {g: fetch_idx(g) for g in range(min(AHEAD + 1, nchunk))}
        gh = {}
        for g in range(min(AHEAD, nchunk)):
            for h in ih[g]:
                h.wait()
            gh[g] = issue(g)
        for g in range(nchunk):
            ng = g + AHEAD
            if ng < nchunk:
                if ng + 1 < nchunk:
                    ih[ng + 1] = fetch_idx(ng + 1)
                for h in ih[ng]:
                    h.wait()
                gh[ng] = issue(ng)
            for h in gh[g]:
                h.wait()
            compute(g)

        for c in range(NUM_CTX):
            pltpu.sync_copy(out_v.at[pl.ds(c, 1), pl.ds(0, items)],
                            out_hbm.at[pl.ds(c, 1), pl.ds(wbase, items)])

    return k


@jax.jit
def kernel(target, context, W_target, W_context):
    B = target.shape[0]
    tgt = target.astype(jnp.int32)
    ctx_t = jnp.transpose(context.astype(jnp.int32))
    dots_t = _make_kernel(B)(tgt, ctx_t, W_target, W_context)
    return jnp.transpose(dots_t)
